# baseline (speedup 1.0000x reference)
import functools

import jax
import jax.numpy as jnp
from jax import lax
from jax.experimental import pallas as pl
from jax.experimental.pallas import tpu as pltpu
from jax.experimental.pallas import tpu_sc as plsc

N = 10000
E = 320000
D = 128
H = 128
EPSBN = 1e-5

C = 128
DH = D // 2
NCHUNK = 2560
E_PAD = NCHUNK * C
CHUNKS_PER_W = NCHUNK // 16
CPB = 16
NBLK = CHUNKS_PER_W // CPB
PAD_ROWS = 8
N_PAD = 10240
ROWS_PER_TILE = N_PAD // 16


def _sc_agg(x2, src2d, dst2d):
    mesh = plsc.VectorSubcoreMesh(core_axis_name="c", subcore_axis_name="s")

    @functools.partial(
        pl.kernel,
        out_type=jax.ShapeDtypeStruct((2, N_PAD, DH), jnp.float32),
        mesh=mesh,
        compiler_params=pltpu.CompilerParams(use_tc_tiling_on_sc=False),
        scratch_types=[
            [pltpu.VMEM((CPB, C), jnp.int32) for _ in range(2)],
            [pltpu.VMEM((CPB, C), jnp.int32) for _ in range(2)],
            [pltpu.VMEM((C, DH), jnp.float32) for _ in range(2)],
            pltpu.VMEM_SHARED((N_PAD, DH), jnp.float32),
            pltpu.VMEM_SHARED((N_PAD, DH), jnp.float32),
            [pltpu.SemaphoreType.DMA for _ in range(2)],
            [pltpu.SemaphoreType.DMA for _ in range(2)],
            pltpu.SemaphoreType.DMA,
            pltpu.SemaphoreType.DMA,
        ],
    )
    def agg_kernel(x_hbm, src_hbm, dst_hbm, out_hbm,
                   isrc, idst, bufs, x_sh, acc_sh, gsems, ssems, isem, xsem):
        cid = lax.axis_index("c")
        sid = lax.axis_index("s")
        base = sid * CHUNKS_PER_W
        r0 = sid * ROWS_PER_TILE

        def fetch_block(b):
            cp = pltpu.make_async_copy(
                src_hbm.at[pl.ds(base + b * CPB, CPB)], isrc[b % 2], isem)
            cp2 = pltpu.make_async_copy(
                dst_hbm.at[pl.ds(base + b * CPB, CPB)], idst[b % 2], isem)
            return cp, cp2

        xcp = pltpu.make_async_copy(x_hbm.at[cid, pl.ds(r0, ROWS_PER_TILE)],
                                    x_sh.at[pl.ds(r0, ROWS_PER_TILE)], xsem)
        xcp.start()
        c1, c2 = fetch_block(0)
        c1.start(); c2.start()

        for k in range(C // PAD_ROWS):
            pltpu.sync_copy(x_hbm.at[cid, pl.ds(N, PAD_ROWS)],
                            bufs[0].at[pl.ds(k * PAD_ROWS, PAD_ROWS)])
        for k in range(ROWS_PER_TILE // C):
            pltpu.sync_copy(bufs[0], acc_sh.at[pl.ds(r0 + k * C, C)])
        c1.wait(); c2.wait(); xcp.wait()
        plsc.subcore_barrier()

        for b in range(NBLK):
            sb, db = isrc[b % 2], idst[b % 2]
            if b + 1 < NBLK:
                n1, n2 = fetch_block(b + 1)
                n1.start(); n2.start()

            for k in range(2):
                pltpu.make_async_copy(x_sh.at[sb.at[k]], bufs[k], gsems[k]).start()

            def body(jj, _):
                j0 = 2 * jj
                scats = []
                for k in range(2):
                    pltpu.make_async_copy(x_sh.at[sb.at[j0 + k]], bufs[k],
                                          gsems[k]).wait()
                    scats.append(pltpu.async_copy(
                        bufs[k], acc_sh.at[db.at[j0 + k]], ssems[k], add=True))
                for k in range(2):
                    scats[k].wait()

                    @pl.when(j0 + k + 2 < CPB)
                    def _():
                        pltpu.make_async_copy(x_sh.at[sb.at[j0 + k + 2]],
                                              bufs[k], gsems[k]).start()

                return _
            lax.fori_loop(0, CPB // 2, body, None)

            if b + 1 < NBLK:
                n1, n2 = fetch_block(b + 1)
                n1.wait(); n2.wait()

        plsc.subcore_barrier()
        pltpu.sync_copy(acc_sh.at[pl.ds(r0, ROWS_PER_TILE)],
                        out_hbm.at[cid, pl.ds(r0, ROWS_PER_TILE)])

    return agg_kernel(x2, src2d, dst2d)


def _mlp_body(x_ref, a_ref, w1_ref, b1_ref, g_ref, be_ref, w2_ref, b2_ref, o_ref):
    agg = jnp.concatenate([a_ref[0, :N], a_ref[1, :N]], axis=1)
    h = x_ref[...] + agg
    h = jnp.dot(h, w1_ref[...], preferred_element_type=jnp.float32) + b1_ref[...]
    mean = jnp.mean(h, axis=0, keepdims=True)
    var = jnp.mean((h - mean) ** 2, axis=0, keepdims=True)
    h = (h - mean) * lax.rsqrt(var + EPSBN) * g_ref[...] + be_ref[...]
    h = jnp.maximum(h, 0.0)
    h = jnp.dot(h, w2_ref[...], preferred_element_type=jnp.float32) + b2_ref[...]
    o_ref[...] = jnp.where(h > 0.0, h, jnp.exp(jnp.minimum(h, 0.0)) - 1.0)


def _mlp(x, agg, w1, b1, g, be, w2, b2):
    return pl.pallas_call(
        _mlp_body,
        out_shape=jax.ShapeDtypeStruct((N, H), jnp.float32),
    )(x, agg, w1, b1.reshape(1, H), g.reshape(1, H), be.reshape(1, H),
      w2, b2.reshape(1, H))


def _split_cols(x):
    zpad = jnp.zeros((N_PAD - N, D), jnp.float32)
    xp = jnp.concatenate([x, zpad], axis=0)
    return xp.reshape(N_PAD, 2, DH).swapaxes(0, 1)


def kernel(x, edge_index, W1_0, b1_0, g_0, be_0, W2_0, b2_0,
           W1_1, b1_1, g_1, be_1, W2_1, b2_1):
    src = edge_index[0]
    dst = edge_index[1]
    npad = E_PAD - E
    pad_iota = jnp.arange(npad, dtype=jnp.int32)
    src2d = jnp.concatenate([src, N + (pad_iota % PAD_ROWS)]).reshape(NCHUNK, C)
    dst2d = jnp.concatenate([dst, pad_iota % N]).reshape(NCHUNK, C)

    agg0 = _sc_agg(_split_cols(x), src2d, dst2d)
    h = _mlp(x, agg0, W1_0, b1_0, g_0, be_0, W2_0, b2_0)
    agg1 = _sc_agg(_split_cols(h), src2d, dst2d)
    h = _mlp(h, agg1, W1_1, b1_1, g_1, be_1, W2_1, b2_1)
    return h


# speedup vs baseline: 1.3561x; 1.3561x over previous
import functools

import jax
import jax.numpy as jnp
from jax import lax
from jax.experimental import pallas as pl
from jax.experimental.pallas import tpu as pltpu
from jax.experimental.pallas import tpu_sc as plsc

N = 10000
E = 320000
D = 128
H = 128
EPSBN = 1e-5

C = 128
NWORKERS = 32
NCHUNK = 2560
E_PAD = NCHUNK * C
CHUNKS_PER_W = NCHUNK // NWORKERS
CPB = 16
NBLK = CHUNKS_PER_W // CPB
N_PAD = 10240
ROWS_PER_TILE = N_PAD // 16
ZR = 8


def _sc_agg(x, src2d, dst2d, zseed):
    mesh = plsc.VectorSubcoreMesh(core_axis_name="c", subcore_axis_name="s")

    @functools.partial(
        pl.kernel,
        out_type=jax.ShapeDtypeStruct((2, N_PAD, D), jnp.float32),
        mesh=mesh,
        scratch_types=[
            [pltpu.VMEM((CPB, C), jnp.int32) for _ in range(2)],
            [pltpu.VMEM((CPB, C), jnp.int32) for _ in range(2)],
            [pltpu.VMEM((C, D), jnp.float32) for _ in range(2)],
            pltpu.VMEM_SHARED((N_PAD, D), jnp.float32),
            [pltpu.SemaphoreType.DMA for _ in range(2)],
            [pltpu.SemaphoreType.DMA for _ in range(2)],
            pltpu.SemaphoreType.DMA,
        ],
    )
    def agg_kernel(x_hbm, src_hbm, dst_hbm, z_hbm, out_hbm,
                   isrc, idst, bufs, acc_sh, gsems, ssems, isem):
        cid = lax.axis_index("c")
        sid = lax.axis_index("s")
        wid = sid * 2 + cid
        base = wid * CHUNKS_PER_W
        r0 = sid * ROWS_PER_TILE

        def fetch_block(b):
            cp = pltpu.make_async_copy(
                src_hbm.at[pl.ds(base + b * CPB, CPB)], isrc[b % 2], isem)
            cp2 = pltpu.make_async_copy(
                dst_hbm.at[pl.ds(base + b * CPB, CPB)], idst[b % 2], isem)
            return cp, cp2

        c1, c2 = fetch_block(0)
        c1.start(); c2.start()
        for k in range(C // ZR):
            pltpu.sync_copy(z_hbm, bufs[0].at[pl.ds(k * ZR, ZR)])
        for k in range(ROWS_PER_TILE // C):
            pltpu.sync_copy(bufs[0], acc_sh.at[pl.ds(r0 + k * C, C)])
        c1.wait(); c2.wait()
        plsc.subcore_barrier()

        for b in range(NBLK):
            sb, db = isrc[b % 2], idst[b % 2]
            if b + 1 < NBLK:
                n1, n2 = fetch_block(b + 1)
                n1.start(); n2.start()

            for k in range(2):
                pltpu.make_async_copy(x_hbm.at[sb.at[k]], bufs[k], gsems[k]).start()

            def body(jj, _):
                j0 = 2 * jj
                scats = []
                for k in range(2):
                    pltpu.make_async_copy(x_hbm.at[sb.at[j0 + k]], bufs[k],
                                          gsems[k]).wait()
                    scats.append(pltpu.async_copy(
                        bufs[k], acc_sh.at[db.at[j0 + k]], ssems[k], add=True))
                for k in range(2):
                    scats[k].wait()

                    @pl.when(j0 + k + 2 < CPB)
                    def _():
                        pltpu.make_async_copy(x_hbm.at[sb.at[j0 + k + 2]],
                                              bufs[k], gsems[k]).start()

                return _
            lax.fori_loop(0, CPB // 2, body, None)

            if b + 1 < NBLK:
                n1, n2 = fetch_block(b + 1)
                n1.wait(); n2.wait()

        plsc.subcore_barrier()
        pltpu.sync_copy(acc_sh.at[pl.ds(r0, ROWS_PER_TILE)],
                        out_hbm.at[cid, pl.ds(r0, ROWS_PER_TILE)])

    return agg_kernel(x, src2d, dst2d, zseed)


def _mlp_body(x_ref, a_ref, w1_ref, b1_ref, g_ref, be_ref, w2_ref, b2_ref, o_ref):
    h = x_ref[...] + a_ref[0, :N] + a_ref[1, :N]
    h = jnp.dot(h, w1_ref[...], preferred_element_type=jnp.float32) + b1_ref[...]
    mean = jnp.mean(h, axis=0, keepdims=True)
    var = jnp.mean((h - mean) ** 2, axis=0, keepdims=True)
    h = (h - mean) * lax.rsqrt(var + EPSBN) * g_ref[...] + be_ref[...]
    h = jnp.maximum(h, 0.0)
    h = jnp.dot(h, w2_ref[...], preferred_element_type=jnp.float32) + b2_ref[...]
    o_ref[...] = jnp.where(h > 0.0, h, jnp.exp(jnp.minimum(h, 0.0)) - 1.0)


def _mlp(x, agg, w1, b1, g, be, w2, b2):
    return pl.pallas_call(
        _mlp_body,
        out_shape=jax.ShapeDtypeStruct((N, H), jnp.float32),
    )(x, agg, w1, b1.reshape(1, H), g.reshape(1, H), be.reshape(1, H),
      w2, b2.reshape(1, H))


def kernel(x, edge_index, W1_0, b1_0, g_0, be_0, W2_0, b2_0,
           W1_1, b1_1, g_1, be_1, W2_1, b2_1):
    src = edge_index[0]
    dst = edge_index[1]
    npad = E_PAD - E
    pad_iota = jnp.arange(npad, dtype=jnp.int32)
    src2d = jnp.concatenate([src, pad_iota % N]).reshape(NCHUNK, C)
    dst2d = jnp.concatenate([dst, N + pad_iota % (N_PAD - N)]).reshape(NCHUNK, C)
    zseed = jnp.zeros((ZR, D), jnp.float32)

    agg0 = _sc_agg(x, src2d, dst2d, zseed)
    h = _mlp(x, agg0, W1_0, b1_0, g_0, be_0, W2_0, b2_0)
    agg1 = _sc_agg(h, src2d, dst2d, zseed)
    h = _mlp(h, agg1, W1_1, b1_1, g_1, be_1, W2_1, b2_1)
    return h
